# Initial kernel scaffold; baseline (speedup 1.0000x reference)
import jax
import jax.numpy as jnp
from jax.experimental import pallas as pl


def kernel(x, edge_index, root_node_indices, W1, b1, W2, b2, Wh1, bh1, Wh2, bh2, W1_off, b1_off, W2_off, b2_off):
    raise NotImplementedError("write your pallas kernel here")


# speedup vs baseline: 9.6417x; 9.6417x over previous
import functools

import numpy as np
import jax
import jax.numpy as jnp
from jax import lax
from jax.experimental import pallas as pl
from jax.experimental.pallas import tpu as pltpu
from jax.experimental.pallas import tpu_sc as plsc

_N = 10000
_E = 320000
_D = 128
_B = 1024
_NEG_LAMBDA = 0.12
_FD = (0.8, 0.1, 0.95)
_ED = (0.8, 0.8, 0.95)

_NTILES = 32
_NSUB = 16
_CHUNK = 128
_NPAD = 10112
_ROWS_PER_SUB = _NPAD // _NSUB
_ZROWS = 79
_RPT = _B // _NTILES
_RPS = _B // _NSUB


@functools.lru_cache(maxsize=None)
def _aug_consts():
    cpu = jax.local_devices(backend="cpu")[0]
    with jax.set_mesh(None), jax.default_device(cpu):
        return _aug_consts_impl()


def _aug_consts_impl():
    akey = jax.random.key(42)
    fms = []
    kept = []
    perm = None
    for a in (1, 2, 3):
        kf, ke, kp = jax.random.split(jax.random.fold_in(akey, a), 3)
        fm = (jax.random.uniform(kf, (_N, _D)) > _FD[a - 1]).astype(jnp.float32)
        em = jax.random.uniform(ke, (_E,)) > _ED[a - 1]
        if a == 3:
            perm = np.asarray(jax.random.permutation(kp, _N)).astype(np.int32)
        fms.append(np.asarray(fm))
        kept.append(np.nonzero(np.asarray(em))[0].astype(np.int32))
    keptc = []
    nchs = []
    for k in kept:
        kt = -(-len(k) // (_NTILES * _CHUNK)) * _CHUNK
        buf = np.full((_NTILES * kt,), _E, np.int32)
        buf[: len(k)] = k
        keptc.append(buf.reshape(_NTILES * (kt // _CHUNK), _CHUNK))
        nchs.append(kt // _CHUNK)
    permpad = np.zeros((_NPAD,), np.int32)
    permpad[:_N] = perm
    return fms, keptc, tuple(nchs), permpad


def _mask_body(x_ref, f1_ref, f2_ref, f3_ref, o1_ref, o2_ref, o3_ref):
    xv = x_ref[...]
    o1_ref[...] = xv * f1_ref[...]
    o2_ref[...] = xv * f2_ref[...]
    o3_ref[...] = xv * f3_ref[...]


def _pass_mask(x, fm1, fm2, fm3):
    grid = pl.cdiv(_N, 128)
    spec = pl.BlockSpec((128, _D), lambda i: (i, 0))
    return pl.pallas_call(
        _mask_body,
        grid=(grid,),
        in_specs=[spec, spec, spec, spec],
        out_specs=[spec, spec, spec],
        out_shape=[jax.ShapeDtypeStruct((_N, _D), jnp.float32)] * 3,
    )(x, fm1, fm2, fm3)


def _zero_vmem_2d(ref, rows):
    z16 = jnp.zeros((16,), jnp.float32)

    def body(i, _):
        r = i // 8
        c = i % 8
        ref[r, pl.ds(c * 16, 16)] = z16
        return 0

    lax.fori_loop(0, rows * 8, body, 0)


def _zero_vmem_1d(ref, n16):
    z16 = jnp.zeros((16,), jnp.float32)

    def body(i, _):
        ref[pl.ds(i * 16, 16)] = z16
        return 0

    lax.fori_loop(0, n16, body, 0)


def _sc_scatter_call(nch, remap, layer1, table, srcp, dstp, keptc, permpad, r2):
    if layer1:
        out_type = (
            jax.ShapeDtypeStruct((2, _NPAD, _D), jnp.float32),
            jax.ShapeDtypeStruct((_NTILES, _NPAD), jnp.float32),
        )
    else:
        out_type = jax.ShapeDtypeStruct((2, _B, _D), jnp.float32)

    scratch = dict(
        kidx_v=pltpu.VMEM((nch, _CHUNK), jnp.int32),
        sval_v=pltpu.VMEM((nch, _CHUNK), jnp.int32),
        dval_v=pltpu.VMEM((nch, _CHUNK), jnp.int32),
        gbuf=pltpu.VMEM((_CHUNK, _D), jnp.float32),
        zbuf=pltpu.VMEM((_ZROWS, _D), jnp.float32),
        acc_sh=pltpu.VMEM_SHARED((_NPAD, _D), jnp.float32),
        sem=pltpu.SemaphoreType.DMA,
        sem2=pltpu.SemaphoreType.DMA,
    )
    if layer1:
        scratch["degv"] = pltpu.VMEM((_NPAD,), jnp.float32)
    if remap:
        scratch["permv"] = pltpu.VMEM((_NPAD,), jnp.int32)
    if not layer1:
        scratch["rv"] = pltpu.VMEM((_RPS,), jnp.int32)
        scratch["rbuf"] = pltpu.VMEM((_RPS, _D), jnp.float32)

    mesh = plsc.VectorSubcoreMesh(core_axis_name="c", subcore_axis_name="s")

    def body(keptc_ref, srcp_ref, dstp_ref, table_ref, *rest):
        args = list(rest)
        perm_ref = args.pop(0) if remap else None
        r2_ref = None if layer1 else args.pop(0)
        if layer1:
            acc_out, deg_out = args.pop(0), args.pop(0)
        else:
            acc_out = args.pop(0)
        sc = dict(zip(sorted(scratch.keys()), args))
        kidx_v = sc["kidx_v"]; sval_v = sc["sval_v"]; dval_v = sc["dval_v"]
        gbuf = sc["gbuf"]; zbuf = sc["zbuf"]; acc_sh = sc["acc_sh"]
        sem = sc["sem"]; sem2 = sc["sem2"]

        cid = lax.axis_index("c")
        sid = lax.axis_index("s")
        wid = sid * 2 + cid

        _zero_vmem_2d(zbuf, _ZROWS)
        for k in range(8):
            pltpu.sync_copy(zbuf, acc_sh.at[pl.ds(sid * _ROWS_PER_SUB + k * _ZROWS, _ZROWS)])

        pltpu.sync_copy(keptc_ref.at[pl.ds(wid * nch, nch)], kidx_v)
        descs = []
        for j in range(nch):
            descs.append(pltpu.async_copy(srcp_ref.at[kidx_v.at[j]], sval_v.at[j], sem))
            descs.append(pltpu.async_copy(dstp_ref.at[kidx_v.at[j]], dval_v.at[j], sem))
        if remap:
            pltpu.sync_copy(perm_ref, sc["permv"])
        for d in descs:
            d.wait()

        if remap:
            permv = sc["permv"]

            def remap_body(i, _):
                j = i // 8
                k = i % 8
                s = sval_v[j, pl.ds(k * 16, 16)]
                sval_v[j, pl.ds(k * 16, 16)] = plsc.load_gather(permv, [s])
                return 0

            lax.fori_loop(0, nch * 8, remap_body, 0)

        if layer1:
            degv = sc["degv"]
            _zero_vmem_1d(degv, _NPAD // 16)
            ones = jnp.ones((16,), jnp.float32)

            def deg_body(i, _):
                j = i // 8
                k = i % 8
                dv = dval_v[j, pl.ds(k * 16, 16)]
                plsc.addupdate_scatter(degv, [dv], ones)
                return 0

            lax.fori_loop(0, nch * 8, deg_body, 0)

        plsc.subcore_barrier()

        for j in range(nch):
            pltpu.async_copy(table_ref.at[sval_v.at[j]], gbuf, sem2).wait()
            pltpu.sync_copy(gbuf, acc_sh.at[dval_v.at[j]], add=True)

        plsc.subcore_barrier()

        lo = sid * _ROWS_PER_SUB
        if layer1:
            pltpu.sync_copy(acc_sh.at[pl.ds(lo, _ROWS_PER_SUB)],
                            acc_out.at[cid, pl.ds(lo, _ROWS_PER_SUB)])
            pltpu.sync_copy(degv, deg_out.at[wid])
        else:
            rv = sc["rv"]; rbuf = sc["rbuf"]
            pltpu.sync_copy(r2_ref.at[sid // 2, pl.ds((sid % 2) * _RPS, _RPS)], rv)
            pltpu.async_copy(acc_sh.at[rv], rbuf, sem2).wait()
            pltpu.sync_copy(rbuf, acc_out.at[cid, pl.ds(sid * _RPS, _RPS)])

    fn = pl.kernel(
        body,
        out_type=out_type,
        mesh=mesh,
        scratch_types=[scratch[k] for k in sorted(scratch.keys())],
        compiler_params=pltpu.CompilerParams(needs_layout_passes=False),
    )
    args = [keptc, srcp, dstp, table]
    if remap:
        args.append(permpad)
    if not layer1:
        args.append(r2)
    return fn(*args)


def _sc_deg_gather(deg3, r2):
    mesh = plsc.VectorSubcoreMesh(core_axis_name="c", subcore_axis_name="s")

    def body(deg3_ref, r2_ref, out_ref, degv, rv, dv):
        cid = lax.axis_index("c")
        sid = lax.axis_index("s")
        wid = sid * 2 + cid

        @pl.when(wid < 3)
        def _():
            pltpu.sync_copy(deg3_ref.at[wid], degv)
            pltpu.sync_copy(r2_ref, rv)

            def gbody(i, _):
                j = i // 8
                k = i % 8
                idx = rv[j, pl.ds(k * 16, 16)]
                dv[j, pl.ds(k * 16, 16)] = plsc.load_gather(degv, [idx])
                return 0

            lax.fori_loop(0, 64, gbody, 0)
            pltpu.sync_copy(dv, out_ref.at[wid])

    fn = pl.kernel(
        body,
        out_type=jax.ShapeDtypeStruct((3, 8, 128), jnp.float32),
        mesh=mesh,
        scratch_types=[
            pltpu.VMEM((_NPAD,), jnp.float32),
            pltpu.VMEM((8, 128), jnp.int32),
            pltpu.VMEM((8, 128), jnp.float32),
        ],
        compiler_params=pltpu.CompilerParams(needs_layout_passes=False),
    )
    return fn(deg3, r2)


def _enc1_body(a1_ref, d1_ref, a2_ref, d2_ref, a3_ref, d3_ref,
               w1_ref, b1_ref, w1o_ref, b1o_ref,
               h1on_ref, h1off_ref, h2on_ref, h2off_ref, h3off_ref, deg_ref):
    w1 = w1_ref[...]
    b1 = b1_ref[...]
    w1o = w1o_ref[...]
    b1o = b1o_ref[...]
    degs = []
    for a_ref, d_ref, on_ref, off_ref in (
        (a1_ref, d1_ref, h1on_ref, h1off_ref),
        (a2_ref, d2_ref, h2on_ref, h2off_ref),
        (a3_ref, d3_ref, None, h3off_ref),
    ):
        deg = jnp.maximum(jnp.sum(d_ref[...], axis=0), 1.0)
        degs.append(deg)
        agg = (a_ref[0, :, :] + a_ref[1, :, :]) / deg[:, None]
        if on_ref is not None:
            on_ref[...] = jnp.maximum(
                jnp.dot(agg, w1, preferred_element_type=jnp.float32) + b1, 0.0)
        off_ref[...] = jnp.maximum(
            jnp.dot(agg, w1o, preferred_element_type=jnp.float32) + b1o, 0.0)
    deg_ref[...] = jnp.stack(degs)


def _pass_enc1(accP, degP, W1, b1, W1_off, b1_off):
    grid = _NPAD // 128
    acc_spec = pl.BlockSpec((2, 128, _D), lambda i: (0, i, 0))
    deg_spec = pl.BlockSpec((_NTILES, 128), lambda i: (0, i))
    w_spec = pl.BlockSpec((_D, _D), lambda i: (0, 0))
    b_spec = pl.BlockSpec((1, _D), lambda i: (0, 0))
    h_spec = pl.BlockSpec((128, _D), lambda i: (i, 0))
    dout_spec = pl.BlockSpec((3, 128), lambda i: (0, i))
    return pl.pallas_call(
        _enc1_body,
        grid=(grid,),
        in_specs=[acc_spec, deg_spec, acc_spec, deg_spec, acc_spec, deg_spec,
                  w_spec, b_spec, w_spec, b_spec],
        out_specs=[h_spec] * 5 + [dout_spec],
        out_shape=[jax.ShapeDtypeStruct((_NPAD, _D), jnp.float32)] * 5
        + [jax.ShapeDtypeStruct((3, _NPAD), jnp.float32)],
    )(accP[0], degP[0], accP[1], degP[1], accP[2], degP[2],
      W1, b1.reshape(1, _D), W1_off, b1_off.reshape(1, _D))


def _cos_mean(a, b):
    na = jnp.sqrt(jnp.sum(a * a, axis=-1, keepdims=True)) + 1e-8
    nb = jnp.sqrt(jnp.sum(b * b, axis=-1, keepdims=True)) + 1e-8
    return jnp.mean(jnp.sum((a / na) * (b / nb), axis=-1))


def _loss_body(p1on_ref, p1off_ref, p2on_ref, p2off_ref, p3off_ref, degr_ref,
               w2_ref, b2_ref, w2o_ref, b2o_ref,
               wh1_ref, bh1_ref, wh2_ref, bh2_ref, out_ref):
    def mm(a, w_ref, b_ref):
        return jnp.dot(a, w_ref[...], preferred_element_type=jnp.float32) + b_ref[...]

    degr = degr_ref[...]
    d1 = degr[0, :][:, None]
    d2 = degr[1, :][:, None]
    d3 = degr[2, :][:, None]
    a1on = (p1on_ref[0, :, :] + p1on_ref[1, :, :]) / d1
    a1off = (p1off_ref[0, :, :] + p1off_ref[1, :, :]) / d1
    a2on = (p2on_ref[0, :, :] + p2on_ref[1, :, :]) / d2
    a2off = (p2off_ref[0, :, :] + p2off_ref[1, :, :]) / d2
    a3off = (p3off_ref[0, :, :] + p3off_ref[1, :, :]) / d3
    enc1 = mm(a1on, w2_ref, b2_ref)
    enc2 = mm(a2on, w2_ref, b2_ref)
    y1 = mm(a1off, w2o_ref, b2o_ref)
    y2 = mm(a2off, w2o_ref, b2o_ref)
    neg_y = mm(a3off, w2o_ref, b2o_ref)
    q1 = mm(jnp.maximum(mm(enc1, wh1_ref, bh1_ref), 0.0), wh2_ref, bh2_ref)
    q2 = mm(jnp.maximum(mm(enc2, wh1_ref, bh1_ref), 0.0), wh2_ref, bh2_ref)
    pos = 0.5 * (_cos_mean(q1, y2) + _cos_mean(q2, y1))
    neg = 0.5 * (_cos_mean(q1, neg_y) + _cos_mean(q2, neg_y))
    loss = (1.0 - _NEG_LAMBDA) * (1.0 - pos) + _NEG_LAMBDA * neg
    out_ref[...] = jnp.reshape(loss, (1, 1))


def _pass_loss(rp, degr, W2, b2, W2_off, b2_off, Wh1, bh1, Wh2, bh2):
    return pl.pallas_call(
        _loss_body,
        out_shape=jax.ShapeDtypeStruct((1, 1), jnp.float32),
    )(rp[0], rp[1], rp[2], rp[3], rp[4], degr,
      W2, b2.reshape(1, _D), W2_off, b2_off.reshape(1, _D),
      Wh1, bh1.reshape(1, _D), Wh2, bh2.reshape(1, _D))


_CONSTS = _aug_consts()


def kernel(x, edge_index, root_node_indices, W1, b1, W2, b2, Wh1, bh1, Wh2,
           bh2, W1_off, b1_off, W2_off, b2_off):
    fms, keptc, nchs, permpad = _CONSTS

    srcp = jnp.concatenate([edge_index[0], jnp.zeros((8,), jnp.int32)])
    dstp = jnp.concatenate([edge_index[1], jnp.full((8,), _N, jnp.int32)])
    r2 = root_node_indices.reshape(8, 128)

    xm1, xm2, xm3 = _pass_mask(x, fms[0], fms[1], fms[2])

    accP, degP = [], []
    for a in range(3):
        table = (xm1, xm2, xm3)[a]
        acc, deg = _sc_scatter_call(nchs[a], a == 2, True, table, srcp, dstp,
                                    keptc[a], permpad, None)
        accP.append(acc)
        degP.append(deg)

    h1on, h1off, h2on, h2off, h3off, deg3 = _pass_enc1(
        accP, degP, W1, b1, W1_off, b1_off)

    rp = []
    for a, table in ((0, h1on), (0, h1off), (1, h2on), (1, h2off), (2, h3off)):
        rp.append(_sc_scatter_call(nchs[a], False, False, table, srcp, dstp,
                                   keptc[a], None, r2))

    degr = _sc_deg_gather(deg3, r2).reshape(3, _B)

    loss = _pass_loss(rp, degr, W2, b2, W2_off, b2_off, Wh1, bh1, Wh2, bh2)
    return loss[0, 0]


# speedup vs baseline: 11.1406x; 1.1555x over previous
import functools

import numpy as np
import jax
import jax.numpy as jnp
from jax import lax
from jax.experimental import pallas as pl
from jax.experimental.pallas import tpu as pltpu
from jax.experimental.pallas import tpu_sc as plsc

_N = 10000
_E = 320000
_D = 128
_B = 1024
_NEG_LAMBDA = 0.12
_FD = (0.8, 0.1, 0.95)
_ED = (0.8, 0.8, 0.95)

_NTILES = 32
_NSUB = 16
_CHUNKS = (128, 128, 64)
_NPAD = 10112
_ROWS_PER_SUB = _NPAD // _NSUB
_ZROWS = 79
_RPT = _B // _NTILES
_RPS = _B // _NSUB


@functools.lru_cache(maxsize=None)
def _aug_consts():
    cpu = jax.local_devices(backend="cpu")[0]
    with jax.set_mesh(None), jax.default_device(cpu):
        return _aug_consts_impl()


def _aug_consts_impl():
    akey = jax.random.key(42)
    fms = []
    kept = []
    perm = None
    for a in (1, 2, 3):
        kf, ke, kp = jax.random.split(jax.random.fold_in(akey, a), 3)
        fm = (jax.random.uniform(kf, (_N, _D)) > _FD[a - 1]).astype(jnp.float32)
        em = jax.random.uniform(ke, (_E,)) > _ED[a - 1]
        if a == 3:
            perm = np.asarray(jax.random.permutation(kp, _N)).astype(np.int32)
        fms.append(np.asarray(fm))
        kept.append(np.nonzero(np.asarray(em))[0].astype(np.int32))
    keptc = []
    nchs = []
    for k, chunk in zip(kept, _CHUNKS):
        kt = -(-len(k) // (_NTILES * chunk)) * chunk
        buf = np.empty((_NTILES, kt), np.int32)
        pieces = np.array_split(k, _NTILES)
        j = 0
        for t, piece in enumerate(pieces):
            buf[t, : len(piece)] = piece
            npad_t = kt - len(piece)
            buf[t, len(piece):] = _E + (np.arange(j, j + npad_t) % 112)
            j += npad_t
        keptc.append(buf.reshape(_NTILES * (kt // chunk), chunk))
        nchs.append(kt // chunk)
    permpad = np.zeros((_NPAD,), np.int32)
    permpad[:_N] = perm
    return fms, keptc, tuple(nchs), permpad


def _mask_body(x_ref, f1_ref, f2_ref, f3_ref, o1_ref, o2_ref, o3_ref):
    xv = x_ref[...]
    o1_ref[...] = xv * f1_ref[...]
    o2_ref[...] = xv * f2_ref[...]
    o3_ref[...] = xv * f3_ref[...]


def _pass_mask(x, fm1, fm2, fm3):
    grid = pl.cdiv(_N, 128)
    spec = pl.BlockSpec((128, _D), lambda i: (i, 0))
    return pl.pallas_call(
        _mask_body,
        grid=(grid,),
        in_specs=[spec, spec, spec, spec],
        out_specs=[spec, spec, spec],
        out_shape=[jax.ShapeDtypeStruct((_N, _D), jnp.float32)] * 3,
    )(x, fm1, fm2, fm3)


def _zero_vmem_3d0(ref, rows):
    z16 = jnp.zeros((16,), jnp.float32)

    def body(i, _):
        r = i // 8
        c = i % 8
        ref[0, r, pl.ds(c * 16, 16)] = z16
        return 0

    lax.fori_loop(0, rows * 8, body, 0)


def _zero_vmem_1d(ref, n16):
    z16 = jnp.zeros((16,), jnp.float32)

    def body(i, _):
        ref[pl.ds(i * 16, 16)] = z16
        return 0

    lax.fori_loop(0, n16, body, 0)


def _sc_scatter_call(nch, chunk, remap, layer1, table, srcp, dstp, keptc,
                     permpad, r2):
    if layer1:
        out_type = (
            jax.ShapeDtypeStruct((2, _NPAD, _D), jnp.float32),
            jax.ShapeDtypeStruct((_NTILES, _NPAD), jnp.float32),
        )
    else:
        out_type = jax.ShapeDtypeStruct((2, _B, _D), jnp.float32)

    scratch = dict(
        kidx_v=pltpu.VMEM((nch, chunk), jnp.int32),
        sval_v=pltpu.VMEM((nch, chunk), jnp.int32),
        dval_v=pltpu.VMEM((nch, chunk), jnp.int32),
        gbuf=pltpu.VMEM((2, chunk, _D), jnp.float32),
        acc_sh=pltpu.VMEM_SHARED((_NPAD, _D), jnp.float32),
        sem=pltpu.SemaphoreType.DMA,
        sem2=pltpu.SemaphoreType.DMA,
        sem3=pltpu.SemaphoreType.DMA,
    )
    if layer1:
        scratch["degv"] = pltpu.VMEM((_NPAD,), jnp.float32)
    if remap:
        scratch["permv"] = pltpu.VMEM((_NPAD,), jnp.int32)
    if not layer1:
        scratch["rv"] = pltpu.VMEM((_RPS,), jnp.int32)
        scratch["rbuf"] = pltpu.VMEM((_RPS, _D), jnp.float32)

    mesh = plsc.VectorSubcoreMesh(core_axis_name="c", subcore_axis_name="s")

    def body(keptc_ref, srcp_ref, dstp_ref, table_ref, *rest):
        args = list(rest)
        perm_ref = args.pop(0) if remap else None
        r2_ref = None if layer1 else args.pop(0)
        if layer1:
            acc_out, deg_out = args.pop(0), args.pop(0)
        else:
            acc_out = args.pop(0)
        sc = dict(zip(sorted(scratch.keys()), args))
        kidx_v = sc["kidx_v"]; sval_v = sc["sval_v"]; dval_v = sc["dval_v"]
        gbuf = sc["gbuf"]; acc_sh = sc["acc_sh"]
        sem = sc["sem"]; gsems = (sc["sem2"], sc["sem3"])

        cid = lax.axis_index("c")
        sid = lax.axis_index("s")
        wid = sid * 2 + cid

        _zero_vmem_3d0(gbuf, chunk)
        nfull, rem = divmod(_ROWS_PER_SUB, chunk)
        zlo = sid * _ROWS_PER_SUB
        for k in range(nfull):
            pltpu.sync_copy(gbuf.at[0], acc_sh.at[pl.ds(zlo + k * chunk, chunk)])
        if rem:
            pltpu.sync_copy(gbuf.at[0, pl.ds(0, rem)],
                            acc_sh.at[pl.ds(zlo + nfull * chunk, rem)])

        pltpu.sync_copy(keptc_ref.at[pl.ds(wid * nch, nch)], kidx_v)
        descs = []
        for j in range(nch):
            descs.append(pltpu.async_copy(srcp_ref.at[kidx_v.at[j]], sval_v.at[j], sem))
            descs.append(pltpu.async_copy(dstp_ref.at[kidx_v.at[j]], dval_v.at[j], sem))
        if remap:
            pltpu.sync_copy(perm_ref, sc["permv"])
        for d in descs:
            d.wait()

        grp = chunk // 16
        if remap:
            permv = sc["permv"]

            def remap_body(i, _):
                j = i // grp
                k = i % grp
                s = sval_v[j, pl.ds(k * 16, 16)]
                sval_v[j, pl.ds(k * 16, 16)] = plsc.load_gather(permv, [s])
                return 0

            lax.fori_loop(0, nch * grp, remap_body, 0)

        if layer1:
            degv = sc["degv"]
            _zero_vmem_1d(degv, _NPAD // 16)
            ones = jnp.ones((16,), jnp.float32)

            def deg_body(i, _):
                j = i // grp
                k = i % grp
                dv = dval_v[j, pl.ds(k * 16, 16)]
                plsc.addupdate_scatter(degv, [dv], ones)
                return 0

            lax.fori_loop(0, nch * grp, deg_body, 0)

        plsc.subcore_barrier()

        d_prev = pltpu.async_copy(
            table_ref.at[sval_v.at[0]], gbuf.at[0], gsems[0])
        for j in range(1, nch):
            d_cur = pltpu.async_copy(
                table_ref.at[sval_v.at[j]], gbuf.at[j % 2], gsems[j % 2])
            d_prev.wait()
            pltpu.sync_copy(gbuf.at[(j - 1) % 2],
                            acc_sh.at[dval_v.at[j - 1]], add=True)
            d_prev = d_cur
        d_prev.wait()
        pltpu.sync_copy(gbuf.at[(nch - 1) % 2],
                        acc_sh.at[dval_v.at[nch - 1]], add=True)

        plsc.subcore_barrier()

        lo = sid * _ROWS_PER_SUB
        if layer1:
            pltpu.sync_copy(acc_sh.at[pl.ds(lo, _ROWS_PER_SUB)],
                            acc_out.at[cid, pl.ds(lo, _ROWS_PER_SUB)])
            pltpu.sync_copy(degv, deg_out.at[wid])
        else:
            rv = sc["rv"]; rbuf = sc["rbuf"]
            pltpu.sync_copy(r2_ref.at[sid // 2, pl.ds((sid % 2) * _RPS, _RPS)], rv)
            pltpu.async_copy(acc_sh.at[rv], rbuf, gsems[0]).wait()
            pltpu.sync_copy(rbuf, acc_out.at[cid, pl.ds(sid * _RPS, _RPS)])

    fn = pl.kernel(
        body,
        out_type=out_type,
        mesh=mesh,
        scratch_types=[scratch[k] for k in sorted(scratch.keys())],
        compiler_params=pltpu.CompilerParams(needs_layout_passes=False),
    )
    args = [keptc, srcp, dstp, table]
    if remap:
        args.append(permpad)
    if not layer1:
        args.append(r2)
    return fn(*args)


def _sc_deg_gather(deg3, r2):
    mesh = plsc.VectorSubcoreMesh(core_axis_name="c", subcore_axis_name="s")

    def body(deg3_ref, r2_ref, out_ref, degv, rv, dv):
        cid = lax.axis_index("c")
        sid = lax.axis_index("s")
        wid = sid * 2 + cid

        @pl.when(wid < 3)
        def _():
            pltpu.sync_copy(deg3_ref.at[wid], degv)
            pltpu.sync_copy(r2_ref, rv)

            def gbody(i, _):
                j = i // 8
                k = i % 8
                idx = rv[j, pl.ds(k * 16, 16)]
                dv[j, pl.ds(k * 16, 16)] = plsc.load_gather(degv, [idx])
                return 0

            lax.fori_loop(0, 64, gbody, 0)
            pltpu.sync_copy(dv, out_ref.at[wid])

    fn = pl.kernel(
        body,
        out_type=jax.ShapeDtypeStruct((3, 8, 128), jnp.float32),
        mesh=mesh,
        scratch_types=[
            pltpu.VMEM((_NPAD,), jnp.float32),
            pltpu.VMEM((8, 128), jnp.int32),
            pltpu.VMEM((8, 128), jnp.float32),
        ],
        compiler_params=pltpu.CompilerParams(needs_layout_passes=False),
    )
    return fn(deg3, r2)


def _enc1_body(a1_ref, d1_ref, a2_ref, d2_ref, a3_ref, d3_ref,
               w1_ref, b1_ref, w1o_ref, b1o_ref,
               h1on_ref, h1off_ref, h2on_ref, h2off_ref, h3off_ref, deg_ref):
    w1 = w1_ref[...]
    b1 = b1_ref[...]
    w1o = w1o_ref[...]
    b1o = b1o_ref[...]
    degs = []
    for a_ref, d_ref, on_ref, off_ref in (
        (a1_ref, d1_ref, h1on_ref, h1off_ref),
        (a2_ref, d2_ref, h2on_ref, h2off_ref),
        (a3_ref, d3_ref, None, h3off_ref),
    ):
        deg = jnp.maximum(jnp.sum(d_ref[...], axis=0), 1.0)
        degs.append(deg)
        agg = (a_ref[0, :, :] + a_ref[1, :, :]) / deg[:, None]
        if on_ref is not None:
            on_ref[...] = jnp.maximum(
                jnp.dot(agg, w1, preferred_element_type=jnp.float32) + b1, 0.0)
        off_ref[...] = jnp.maximum(
            jnp.dot(agg, w1o, preferred_element_type=jnp.float32) + b1o, 0.0)
    deg_ref[...] = jnp.stack(degs)


def _pass_enc1(accP, degP, W1, b1, W1_off, b1_off):
    grid = _NPAD // 128
    acc_spec = pl.BlockSpec((2, 128, _D), lambda i: (0, i, 0))
    deg_spec = pl.BlockSpec((_NTILES, 128), lambda i: (0, i))
    w_spec = pl.BlockSpec((_D, _D), lambda i: (0, 0))
    b_spec = pl.BlockSpec((1, _D), lambda i: (0, 0))
    h_spec = pl.BlockSpec((128, _D), lambda i: (i, 0))
    dout_spec = pl.BlockSpec((3, 128), lambda i: (0, i))
    return pl.pallas_call(
        _enc1_body,
        grid=(grid,),
        in_specs=[acc_spec, deg_spec, acc_spec, deg_spec, acc_spec, deg_spec,
                  w_spec, b_spec, w_spec, b_spec],
        out_specs=[h_spec] * 5 + [dout_spec],
        out_shape=[jax.ShapeDtypeStruct((_NPAD, _D), jnp.float32)] * 5
        + [jax.ShapeDtypeStruct((3, _NPAD), jnp.float32)],
    )(accP[0], degP[0], accP[1], degP[1], accP[2], degP[2],
      W1, b1.reshape(1, _D), W1_off, b1_off.reshape(1, _D))


def _cos_mean(a, b):
    na = jnp.sqrt(jnp.sum(a * a, axis=-1, keepdims=True)) + 1e-8
    nb = jnp.sqrt(jnp.sum(b * b, axis=-1, keepdims=True)) + 1e-8
    return jnp.mean(jnp.sum((a / na) * (b / nb), axis=-1))


def _loss_body(p1on_ref, p1off_ref, p2on_ref, p2off_ref, p3off_ref, degr_ref,
               w2_ref, b2_ref, w2o_ref, b2o_ref,
               wh1_ref, bh1_ref, wh2_ref, bh2_ref, out_ref):
    def mm(a, w_ref, b_ref):
        return jnp.dot(a, w_ref[...], preferred_element_type=jnp.float32) + b_ref[...]

    degr = degr_ref[...]
    d1 = degr[0, :][:, None]
    d2 = degr[1, :][:, None]
    d3 = degr[2, :][:, None]
    a1on = (p1on_ref[0, :, :] + p1on_ref[1, :, :]) / d1
    a1off = (p1off_ref[0, :, :] + p1off_ref[1, :, :]) / d1
    a2on = (p2on_ref[0, :, :] + p2on_ref[1, :, :]) / d2
    a2off = (p2off_ref[0, :, :] + p2off_ref[1, :, :]) / d2
    a3off = (p3off_ref[0, :, :] + p3off_ref[1, :, :]) / d3
    enc1 = mm(a1on, w2_ref, b2_ref)
    enc2 = mm(a2on, w2_ref, b2_ref)
    y1 = mm(a1off, w2o_ref, b2o_ref)
    y2 = mm(a2off, w2o_ref, b2o_ref)
    neg_y = mm(a3off, w2o_ref, b2o_ref)
    q1 = mm(jnp.maximum(mm(enc1, wh1_ref, bh1_ref), 0.0), wh2_ref, bh2_ref)
    q2 = mm(jnp.maximum(mm(enc2, wh1_ref, bh1_ref), 0.0), wh2_ref, bh2_ref)
    pos = 0.5 * (_cos_mean(q1, y2) + _cos_mean(q2, y1))
    neg = 0.5 * (_cos_mean(q1, neg_y) + _cos_mean(q2, neg_y))
    loss = (1.0 - _NEG_LAMBDA) * (1.0 - pos) + _NEG_LAMBDA * neg
    out_ref[...] = jnp.reshape(loss, (1, 1))


def _pass_loss(rp, degr, W2, b2, W2_off, b2_off, Wh1, bh1, Wh2, bh2):
    return pl.pallas_call(
        _loss_body,
        out_shape=jax.ShapeDtypeStruct((1, 1), jnp.float32),
    )(rp[0], rp[1], rp[2], rp[3], rp[4], degr,
      W2, b2.reshape(1, _D), W2_off, b2_off.reshape(1, _D),
      Wh1, bh1.reshape(1, _D), Wh2, bh2.reshape(1, _D))


_CONSTS = _aug_consts()


def kernel(x, edge_index, root_node_indices, W1, b1, W2, b2, Wh1, bh1, Wh2,
           bh2, W1_off, b1_off, W2_off, b2_off):
    fms, keptc, nchs, permpad = _CONSTS

    srcp = jnp.concatenate([edge_index[0], jnp.zeros((112,), jnp.int32)])
    dstp = jnp.concatenate(
        [edge_index[1], jnp.arange(_N, _N + 112, dtype=jnp.int32)])
    r2 = root_node_indices.reshape(8, 128)

    xm1, xm2, xm3 = _pass_mask(x, fms[0], fms[1], fms[2])

    accP, degP = [], []
    for a in range(3):
        table = (xm1, xm2, xm3)[a]
        acc, deg = _sc_scatter_call(nchs[a], _CHUNKS[a], a == 2, True, table,
                                    srcp, dstp, keptc[a], permpad, None)
        accP.append(acc)
        degP.append(deg)

    h1on, h1off, h2on, h2off, h3off, deg3 = _pass_enc1(
        accP, degP, W1, b1, W1_off, b1_off)

    rp = []
    for a, table in ((0, h1on), (0, h1off), (1, h2on), (1, h2off), (2, h3off)):
        rp.append(_sc_scatter_call(nchs[a], _CHUNKS[a], False, False, table,
                                   srcp, dstp, keptc[a], None, r2))

    degr = _sc_deg_gather(deg3, r2).reshape(3, _B)

    loss = _pass_loss(rp, degr, W2, b2, W2_off, b2_off, Wh1, bh1, Wh2, bh2)
    return loss[0, 0]
